# baseline (speedup 1.0000x reference)
import functools

import jax
import jax.numpy as jnp
from jax.experimental import pallas as pl
from jax.experimental.pallas import tpu as pltpu

HIDDEN = 2048
E = 8
DPG = 256
TOKENS = 4096
TM = 256


def _moe_block(x_ref, wg_ref, wd_ref, wu_ref, o_ref):
    xb = x_ref[...]
    s = jax.nn.sigmoid(
        jax.lax.dot_general(xb, wg_ref[...], (((1,), (1,)), ((), ())),
                            preferred_element_type=jnp.float32))
    iota = jax.lax.broadcasted_iota(jnp.int32, (TM, E), 1)
    g1 = jnp.max(s, axis=1, keepdims=True)
    i1 = jnp.min(jnp.where(s == g1, iota, E), axis=1, keepdims=True)
    s2 = jnp.where(iota == i1, -jnp.inf, s)
    g2 = jnp.max(s2, axis=1, keepdims=True)
    i2 = jnp.min(jnp.where(s2 == g2, iota, E), axis=1, keepdims=True)
    c = jnp.where(iota == i1, g1, 0.0) + jnp.where(iota == i2, g2, 0.0)

    t1 = jax.lax.dot_general(xb, wd_ref[...], (((1,), (1,)), ((), ())),
                             preferred_element_type=jnp.float32)
    down = c[:, 0:1] * t1[:, :DPG]
    for e in range(1, E):
        down = down + c[:, e:e + 1] * t1[:, e * DPG:(e + 1) * DPG]

    acc = c[:, 0:1] * jax.lax.dot_general(
        down, wu_ref[0 * DPG:1 * DPG], (((1,), (0,)), ((), ())),
        preferred_element_type=jnp.float32)
    for e in range(1, E):
        ue = jax.lax.dot_general(down, wu_ref[e * DPG:(e + 1) * DPG],
                                 (((1,), (0,)), ((), ())),
                                 preferred_element_type=jnp.float32)
        acc = acc + c[:, e:e + 1] * ue
    o_ref[...] = acc


@jax.jit
def _moe(xf, Wg, Wdf, Wut):
    nblk = TOKENS // TM
    return pl.pallas_call(
        _moe_block,
        grid=(nblk,),
        in_specs=[
            pl.BlockSpec((TM, HIDDEN), lambda i: (i, 0)),
            pl.BlockSpec((E, HIDDEN), lambda i: (0, 0)),
            pl.BlockSpec((E * DPG, HIDDEN), lambda i: (0, 0)),
            pl.BlockSpec((E * DPG, HIDDEN), lambda i: (0, 0)),
        ],
        out_specs=pl.BlockSpec((TM, HIDDEN), lambda i: (i, 0)),
        out_shape=jax.ShapeDtypeStruct((TOKENS, HIDDEN), jnp.float32),
        compiler_params=pltpu.CompilerParams(
            dimension_semantics=("arbitrary",),
        ),
    )(xf, Wg, Wdf, Wut)


def kernel(x, Wg, Wd, Wu):
    b, l, d = x.shape
    xf = x.reshape(-1, d)
    Wdf = Wd.reshape(E * DPG, HIDDEN)
    Wut = jnp.transpose(Wu, (0, 2, 1)).reshape(E * DPG, HIDDEN)
    out = _moe(xf, Wg, Wdf, Wut)
    return out.reshape(b, l, d)


# speedup vs baseline: 1.3365x; 1.3365x over previous
import functools

import jax
import jax.numpy as jnp
from jax.experimental import pallas as pl
from jax.experimental.pallas import tpu as pltpu

HIDDEN = 2048
E = 8
DPG = 256
TOKENS = 4096
TM = 256


def _moe_block(x_ref, wg_ref, wd_ref, wu_ref, o_ref):
    xb = x_ref[...]
    s = jax.nn.sigmoid(
        jax.lax.dot_general(xb, wg_ref[...], (((1,), (1,)), ((), ())),
                            preferred_element_type=jnp.float32))
    iota = jax.lax.broadcasted_iota(jnp.int32, (TM, E), 1)
    g1 = jnp.max(s, axis=1, keepdims=True)
    i1 = jnp.min(jnp.where(s == g1, iota, E), axis=1, keepdims=True)
    s2 = jnp.where(iota == i1, -jnp.inf, s)
    g2 = jnp.max(s2, axis=1, keepdims=True)
    i2 = jnp.min(jnp.where(s2 == g2, iota, E), axis=1, keepdims=True)
    c = jnp.where(iota == i1, g1, 0.0) + jnp.where(iota == i2, g2, 0.0)

    t1 = jax.lax.dot_general(xb, wd_ref[...], (((1,), (1,)), ((), ())),
                             preferred_element_type=jnp.float32)
    down = c[:, 0:1] * t1[:, :DPG]
    for e in range(1, E):
        down = down + c[:, e:e + 1] * t1[:, e * DPG:(e + 1) * DPG]

    acc = jnp.zeros((TM, HIDDEN), jnp.float32)
    for e in range(E):
        ue = jax.lax.dot_general(down, wu_ref[e], (((1,), (1,)), ((), ())),
                                 preferred_element_type=jnp.float32)
        acc = acc + c[:, e:e + 1] * ue
    o_ref[...] = acc


@jax.jit
def _moe(xf, Wg, Wdf, Wut):
    nblk = TOKENS // TM
    return pl.pallas_call(
        _moe_block,
        grid=(nblk,),
        in_specs=[
            pl.BlockSpec((TM, HIDDEN), lambda i: (i, 0)),
            pl.BlockSpec((E, HIDDEN), lambda i: (0, 0)),
            pl.BlockSpec((E * DPG, HIDDEN), lambda i: (0, 0)),
            pl.BlockSpec((E, HIDDEN, DPG), lambda i: (0, 0, 0)),
        ],
        out_specs=pl.BlockSpec((TM, HIDDEN), lambda i: (i, 0)),
        out_shape=jax.ShapeDtypeStruct((TOKENS, HIDDEN), jnp.float32),
        compiler_params=pltpu.CompilerParams(
            dimension_semantics=("arbitrary",),
        ),
    )(xf, Wg, Wdf, Wut)


def kernel(x, Wg, Wd, Wu):
    b, l, d = x.shape
    xf = x.reshape(-1, d)
    Wdf = Wd.reshape(E * DPG, HIDDEN)
    out = _moe(xf, Wg, Wdf, Wu)
    return out.reshape(b, l, d)


# speedup vs baseline: 1.3752x; 1.0290x over previous
import functools

import jax
import jax.numpy as jnp
from jax.experimental import pallas as pl
from jax.experimental.pallas import tpu as pltpu

HIDDEN = 2048
E = 8
DPG = 256
TOKENS = 4096
TM = 512


def _moe_block(x_ref, wg_ref, wd_ref, wu_ref, o_ref):
    xb = x_ref[...]
    s = jax.nn.sigmoid(
        jax.lax.dot_general(xb, wg_ref[...], (((1,), (1,)), ((), ())),
                            preferred_element_type=jnp.float32))
    iota = jax.lax.broadcasted_iota(jnp.int32, (TM, E), 1)
    g1 = jnp.max(s, axis=1, keepdims=True)
    i1 = jnp.min(jnp.where(s == g1, iota, E), axis=1, keepdims=True)
    s2 = jnp.where(iota == i1, -jnp.inf, s)
    g2 = jnp.max(s2, axis=1, keepdims=True)
    i2 = jnp.min(jnp.where(s2 == g2, iota, E), axis=1, keepdims=True)
    c = jnp.where(iota == i1, g1, 0.0) + jnp.where(iota == i2, g2, 0.0)

    t1 = jax.lax.dot_general(xb, wd_ref[...], (((1,), (1,)), ((), ())),
                             preferred_element_type=jnp.float32)
    down = c[:, 0:1] * t1[:, :DPG]
    for e in range(1, E):
        down = down + c[:, e:e + 1] * t1[:, e * DPG:(e + 1) * DPG]

    acc = jnp.zeros((TM, HIDDEN), jnp.float32)
    for e in range(E):
        ue = jax.lax.dot_general(down, wu_ref[e], (((1,), (1,)), ((), ())),
                                 preferred_element_type=jnp.float32)
        acc = acc + c[:, e:e + 1] * ue
    o_ref[...] = acc


@jax.jit
def _moe(xf, Wg, Wdf, Wut):
    nblk = TOKENS // TM
    return pl.pallas_call(
        _moe_block,
        grid=(nblk,),
        in_specs=[
            pl.BlockSpec((TM, HIDDEN), lambda i: (i, 0)),
            pl.BlockSpec((E, HIDDEN), lambda i: (0, 0)),
            pl.BlockSpec((E * DPG, HIDDEN), lambda i: (0, 0)),
            pl.BlockSpec((E, HIDDEN, DPG), lambda i: (0, 0, 0)),
        ],
        out_specs=pl.BlockSpec((TM, HIDDEN), lambda i: (i, 0)),
        out_shape=jax.ShapeDtypeStruct((TOKENS, HIDDEN), jnp.float32),
        compiler_params=pltpu.CompilerParams(
            dimension_semantics=("arbitrary",),
        ),
    )(xf, Wg, Wdf, Wut)


def kernel(x, Wg, Wd, Wu):
    b, l, d = x.shape
    xf = x.reshape(-1, d)
    Wdf = Wd.reshape(E * DPG, HIDDEN)
    out = _moe(xf, Wg, Wdf, Wu)
    return out.reshape(b, l, d)
